# baseline (speedup 1.0000x reference)
import functools

import jax
import jax.numpy as jnp
from jax import lax
from jax.experimental import pallas as pl
from jax.experimental.pallas import tpu as pltpu
from jax.experimental.pallas import tpu_sc as plsc

_RATE = 0.75
_MIN32 = -(2**31)


def _sc_body(
    k, n, chunk,
    imp_hbm, mask_hbm, cnt_hbm,
    buf0, buf1, ob0, ob1, hist, cbuf,
    si0, si1, so0, so1,
):
    b = lax.axis_index("s") * 2 + lax.axis_index("c")
    lane = lax.iota(jnp.int32, 16)
    ones = jnp.ones((16,), jnp.int32)
    nch = n // chunk
    npair = nch // 2
    nvec = chunk // 16

    def in_copy(c, buf, sem):
        return pltpu.async_copy(imp_hbm.at[b, pl.ds(c * chunk, chunk)], buf, sem)

    def keys_at(buf, j):
        x = buf[pl.ds(j * 16, 16)]
        i32 = lax.bitcast_convert_type(x, jnp.int32)
        return i32 ^ ((i32 >> 31) & jnp.int32(0x7FFFFFFF))

    prefix = jnp.int32(0)
    k_rem = jnp.int32(k)
    for p in range(4):
        shift_b = 24 - 8 * p

        def zero_body(i, _):
            hist[i] = jnp.zeros((16,), jnp.int32)
            return 0

        lax.fori_loop(0, 256, zero_body, 0, unroll=8)

        def hist_chunk(buf):
            def vec_body(j, _, buf=buf, p=p, shift_b=shift_b):
                ukey = keys_at(buf, j) ^ jnp.int32(_MIN32)
                if shift_b:
                    bucket = lax.shift_right_logical(ukey, shift_b) & 0xFF
                else:
                    bucket = ukey & 0xFF
                if p == 0:
                    plsc.addupdate_scatter(hist, [bucket, lane], ones)
                else:
                    hi = lax.shift_right_logical(ukey, shift_b + 8)
                    plsc.addupdate_scatter(
                        hist, [bucket, lane], ones, mask=hi == prefix
                    )
                return 0

            lax.fori_loop(0, nvec, vec_body, 0, unroll=8)

        in_copy(0, buf0, si0)

        def pair_body(g, _):
            c0 = g * 2
            in_copy(c0 + 1, buf1, si1)
            pltpu.make_async_copy(
                imp_hbm.at[b, pl.ds(c0 * chunk, chunk)], buf0, si0
            ).wait()
            hist_chunk(buf0)

            @pl.when(g + 1 < npair)
            def _():
                in_copy(c0 + 2, buf0, si0)

            pltpu.make_async_copy(
                imp_hbm.at[b, pl.ds((c0 + 1) * chunk, chunk)], buf1, si1
            ).wait()
            hist_chunk(buf1)
            return 0

        lax.fori_loop(0, npair, pair_body, 0)

        def scan_body(i, carry):
            cum, bstar, cabove = carry
            bi = 255 - i
            s = jnp.sum(hist[bi])
            newcum = cum + s
            hit = (cum < k_rem) & (newcum >= k_rem)
            return (
                newcum,
                jnp.where(hit, bi, bstar),
                jnp.where(hit, cum, cabove),
            )

        _, bstar, cabove = lax.fori_loop(
            0, 256, scan_body, (jnp.int32(0), jnp.int32(0), jnp.int32(0)),
            unroll=4,
        )
        prefix = (prefix << 8) | bstar
        k_rem = k_rem - cabove

    t_key = prefix ^ jnp.int32(_MIN32)
    need = k_rem

    def mask_chunk(buf, ob, carry):
        def mask_body(j, carry, buf=buf, ob=ob):
            rank_c, cnt_c = carry
            key = keys_at(buf, j)
            gt = key > t_key
            tie = key == t_key
            tcnt = plsc.all_reduce_population_count(tie)

            def slow(_):
                cs = plsc.cumsum(jnp.where(tie, jnp.int32(1), jnp.int32(0)))
                return gt | (tie & ((cs + rank_c) <= need))

            def fast(_):
                return gt

            keep = lax.cond(tcnt[0] > 0, slow, fast, 0)
            ob[pl.ds(j * 16, 16)] = jnp.where(
                keep, jnp.float32(1.0), jnp.float32(0.0)
            )
            return (
                rank_c + tcnt,
                cnt_c + plsc.all_reduce_population_count(keep),
            )

        return lax.fori_loop(0, nvec, mask_body, carry, unroll=4)

    def out_copy(c, ob, sem):
        return pltpu.async_copy(ob, mask_hbm.at[b, pl.ds(c * chunk, chunk)], sem)

    def out_wait(c, ob, sem):
        pltpu.make_async_copy(ob, mask_hbm.at[b, pl.ds(c * chunk, chunk)], sem).wait()

    in_copy(0, buf0, si0)

    def mask_pair_body(g, carry):
        c0 = g * 2
        in_copy(c0 + 1, buf1, si1)
        pltpu.make_async_copy(
            imp_hbm.at[b, pl.ds(c0 * chunk, chunk)], buf0, si0
        ).wait()

        @pl.when(g >= 1)
        def _():
            out_wait(c0 - 2, ob0, so0)

        carry = mask_chunk(buf0, ob0, carry)
        out_copy(c0, ob0, so0)

        @pl.when(g + 1 < npair)
        def _():
            in_copy(c0 + 2, buf0, si0)

        pltpu.make_async_copy(
            imp_hbm.at[b, pl.ds((c0 + 1) * chunk, chunk)], buf1, si1
        ).wait()

        @pl.when(g >= 1)
        def _():
            out_wait(c0 - 1, ob1, so1)

        carry = mask_chunk(buf1, ob1, carry)
        out_copy(c0 + 1, ob1, so1)
        return carry

    rank_c, cnt_c = lax.fori_loop(
        0, npair, mask_pair_body,
        (jnp.zeros((16,), jnp.int32), jnp.zeros((16,), jnp.int32)),
    )
    out_wait(nch - 2, ob0, so0)
    out_wait(nch - 1, ob1, so1)

    cbuf[pl.ds(0, 16)] = cnt_c.astype(jnp.float32)
    pltpu.sync_copy(cbuf, cnt_hbm.at[b])


@jax.jit
def kernel(imp):
    B, H, W = imp.shape
    n = H * W
    k = max(1, int(_RATE * n))
    chunk = 16384
    mesh = plsc.VectorSubcoreMesh(core_axis_name="c", subcore_axis_name="s")
    sc_call = pl.kernel(
        functools.partial(_sc_body, k, n, chunk),
        out_type=[
            jax.ShapeDtypeStruct((B, n), jnp.float32),
            jax.ShapeDtypeStruct((B, 16), jnp.float32),
        ],
        mesh=mesh,
        compiler_params=pltpu.CompilerParams(needs_layout_passes=False),
        scratch_types=[
            pltpu.VMEM((chunk,), jnp.float32),
            pltpu.VMEM((chunk,), jnp.float32),
            pltpu.VMEM((chunk,), jnp.float32),
            pltpu.VMEM((chunk,), jnp.float32),
            pltpu.VMEM((256, 16), jnp.int32),
            pltpu.VMEM((16,), jnp.float32),
            pltpu.SemaphoreType.DMA,
            pltpu.SemaphoreType.DMA,
            pltpu.SemaphoreType.DMA,
            pltpu.SemaphoreType.DMA,
        ],
    )
    mask2d, cnt = sc_call(imp.reshape(B, n))
    mean = jnp.sum(cnt[:, 0]) / jnp.float32(B * n)
    return mask2d.reshape(B, 1, H, W), mean


# speedup vs baseline: 4.6026x; 4.6026x over previous
import functools

import jax
import jax.numpy as jnp
from jax import lax
from jax.experimental import pallas as pl
from jax.experimental.pallas import tpu as pltpu
from jax.experimental.pallas import tpu_sc as plsc

_RATE = 0.75
_MIN32 = -(2**31)


def _sc_body(
    k, n, chunk,
    imp_hbm, mask_hbm, cnt_hbm,
    buf0, buf1, ob0, ob1, hist, cbuf,
    si0, si1, so0, so1,
):
    b = lax.axis_index("s") * 2 + lax.axis_index("c")
    lane = lax.iota(jnp.int32, 16)
    ones = jnp.ones((16,), jnp.int32)
    nch = n // chunk
    npair = nch // 2
    nvec = chunk // 16

    def in_copy(c, buf, sem):
        return pltpu.async_copy(imp_hbm.at[b, pl.ds(c * chunk, chunk)], buf, sem)

    def keys_at(buf, j):
        x = buf[pl.ds(j * 16, 16)]
        i32 = lax.bitcast_convert_type(x, jnp.int32)
        return i32 ^ ((i32 >> 31) & jnp.int32(0x7FFFFFFF))

    prefix = jnp.int32(0)
    k_rem = jnp.int32(k)
    for p in range(4):
        shift_b = 24 - 8 * p

        def zero_body(i, _):
            hist[i] = jnp.zeros((16,), jnp.int32)
            return 0

        lax.fori_loop(0, 256, zero_body, 0, unroll=8)

        def hist_chunk(buf):
            @plsc.parallel_loop(0, nvec, unroll=8)
            def _(j, buf=buf, p=p, shift_b=shift_b):
                ukey = keys_at(buf, j) ^ jnp.int32(_MIN32)
                if shift_b:
                    bucket = lax.shift_right_logical(ukey, shift_b) & 0xFF
                else:
                    bucket = ukey & 0xFF
                if p == 0:
                    plsc.addupdate_scatter(hist, [bucket, lane], ones)
                else:
                    hi = lax.shift_right_logical(ukey, shift_b + 8)
                    plsc.addupdate_scatter(
                        hist, [bucket, lane], ones, mask=hi == prefix
                    )

        in_copy(0, buf0, si0)

        def pair_body(g, _):
            c0 = g * 2
            in_copy(c0 + 1, buf1, si1)
            pltpu.make_async_copy(
                imp_hbm.at[b, pl.ds(c0 * chunk, chunk)], buf0, si0
            ).wait()
            hist_chunk(buf0)

            @pl.when(g + 1 < npair)
            def _():
                in_copy(c0 + 2, buf0, si0)

            pltpu.make_async_copy(
                imp_hbm.at[b, pl.ds((c0 + 1) * chunk, chunk)], buf1, si1
            ).wait()
            hist_chunk(buf1)
            return 0

        lax.fori_loop(0, npair, pair_body, 0)

        def scan_body(i, carry):
            cum, bstar, cabove = carry
            bi = 255 - i
            s = jnp.sum(hist[bi])
            newcum = cum + s
            hit = (cum < k_rem) & (newcum >= k_rem)
            return (
                newcum,
                jnp.where(hit, bi, bstar),
                jnp.where(hit, cum, cabove),
            )

        _, bstar, cabove = lax.fori_loop(
            0, 256, scan_body, (jnp.int32(0), jnp.int32(0), jnp.int32(0)),
            unroll=4,
        )
        prefix = (prefix << 8) | bstar
        k_rem = k_rem - cabove

    t_key = prefix ^ jnp.int32(_MIN32)
    need = k_rem

    def mask_chunk(buf, ob, carry):
        def mask_body(j, carry, buf=buf, ob=ob):
            rank_c, cnt_c = carry
            key = keys_at(buf, j)
            gt = key > t_key
            tie = key == t_key
            cs = plsc.cumsum(jnp.where(tie, jnp.int32(1), jnp.int32(0)))
            keep = gt | (tie & ((cs + rank_c) <= need))
            ob[pl.ds(j * 16, 16)] = jnp.where(
                keep, jnp.float32(1.0), jnp.float32(0.0)
            )
            return (
                rank_c + plsc.all_reduce_population_count(tie),
                cnt_c + plsc.all_reduce_population_count(keep),
            )

        return plsc.parallel_loop(0, nvec, unroll=4, carry=carry)(mask_body)

    def out_copy(c, ob, sem):
        return pltpu.async_copy(ob, mask_hbm.at[b, pl.ds(c * chunk, chunk)], sem)

    def out_wait(c, ob, sem):
        pltpu.make_async_copy(ob, mask_hbm.at[b, pl.ds(c * chunk, chunk)], sem).wait()

    in_copy(0, buf0, si0)

    def mask_pair_body(g, carry):
        c0 = g * 2
        in_copy(c0 + 1, buf1, si1)
        pltpu.make_async_copy(
            imp_hbm.at[b, pl.ds(c0 * chunk, chunk)], buf0, si0
        ).wait()

        @pl.when(g >= 1)
        def _():
            out_wait(c0 - 2, ob0, so0)

        carry = mask_chunk(buf0, ob0, carry)
        out_copy(c0, ob0, so0)

        @pl.when(g + 1 < npair)
        def _():
            in_copy(c0 + 2, buf0, si0)

        pltpu.make_async_copy(
            imp_hbm.at[b, pl.ds((c0 + 1) * chunk, chunk)], buf1, si1
        ).wait()

        @pl.when(g >= 1)
        def _():
            out_wait(c0 - 1, ob1, so1)

        carry = mask_chunk(buf1, ob1, carry)
        out_copy(c0 + 1, ob1, so1)
        return carry

    rank_c, cnt_c = lax.fori_loop(
        0, npair, mask_pair_body,
        (jnp.zeros((16,), jnp.int32), jnp.zeros((16,), jnp.int32)),
    )
    out_wait(nch - 2, ob0, so0)
    out_wait(nch - 1, ob1, so1)

    cbuf[pl.ds(0, 16)] = cnt_c.astype(jnp.float32)
    pltpu.sync_copy(cbuf, cnt_hbm.at[b])


@jax.jit
def kernel(imp):
    B, H, W = imp.shape
    n = H * W
    k = max(1, int(_RATE * n))
    chunk = 16384
    mesh = plsc.VectorSubcoreMesh(core_axis_name="c", subcore_axis_name="s")
    sc_call = pl.kernel(
        functools.partial(_sc_body, k, n, chunk),
        out_type=[
            jax.ShapeDtypeStruct((B, n), jnp.float32),
            jax.ShapeDtypeStruct((B, 16), jnp.float32),
        ],
        mesh=mesh,
        compiler_params=pltpu.CompilerParams(needs_layout_passes=False),
        scratch_types=[
            pltpu.VMEM((chunk,), jnp.float32),
            pltpu.VMEM((chunk,), jnp.float32),
            pltpu.VMEM((chunk,), jnp.float32),
            pltpu.VMEM((chunk,), jnp.float32),
            pltpu.VMEM((256, 16), jnp.int32),
            pltpu.VMEM((16,), jnp.float32),
            pltpu.SemaphoreType.DMA,
            pltpu.SemaphoreType.DMA,
            pltpu.SemaphoreType.DMA,
            pltpu.SemaphoreType.DMA,
        ],
    )
    mask2d, cnt = sc_call(imp.reshape(B, n))
    mean = jnp.sum(cnt[:, 0]) / jnp.float32(B * n)
    return mask2d.reshape(B, 1, H, W), mean


# speedup vs baseline: 4.6417x; 1.0085x over previous
import functools

import jax
import jax.numpy as jnp
from jax import lax
from jax.experimental import pallas as pl
from jax.experimental.pallas import tpu as pltpu
from jax.experimental.pallas import tpu_sc as plsc

_RATE = 0.75
_MIN32 = -(2**31)


def _sc_body(
    k, n, chunk,
    imp_hbm, mask_hbm, cnt_hbm,
    buf0, buf1, ob0, ob1, hist, cbuf,
    si0, si1, so0, so1,
):
    b = lax.axis_index("s") * 2 + lax.axis_index("c")
    lane = lax.iota(jnp.int32, 16)
    ones = jnp.ones((16,), jnp.int32)
    nch = n // chunk
    npair = nch // 2
    nvec = chunk // 16

    def in_copy(c, buf, sem):
        return pltpu.async_copy(imp_hbm.at[b, pl.ds(c * chunk, chunk)], buf, sem)

    def keys_at(buf, j):
        x = buf[pl.ds(j * 16, 16)]
        i32 = lax.bitcast_convert_type(x, jnp.int32)
        return i32 ^ ((i32 >> 31) & jnp.int32(0x7FFFFFFF))

    prefix = jnp.int32(0)
    k_rem = jnp.int32(k)
    for p in range(4):
        shift_b = 24 - 8 * p

        def zero_body(i, _):
            hist[i] = jnp.zeros((16,), jnp.int32)
            return 0

        lax.fori_loop(0, 256, zero_body, 0, unroll=8)

        def hist_chunk(buf):
            @plsc.parallel_loop(0, nvec, unroll=8)
            def _(j, buf=buf, p=p, shift_b=shift_b):
                ukey = keys_at(buf, j) ^ jnp.int32(_MIN32)
                if shift_b:
                    bucket = lax.shift_right_logical(ukey, shift_b) & 0xFF
                else:
                    bucket = ukey & 0xFF
                if p == 0:
                    plsc.addupdate_scatter(hist, [bucket, lane], ones)
                else:
                    hi = lax.shift_right_logical(ukey, shift_b + 8)
                    plsc.addupdate_scatter(
                        hist, [bucket, lane], ones, mask=hi == prefix
                    )

        in_copy(0, buf0, si0)

        def pair_body(g, _):
            c0 = g * 2
            in_copy(c0 + 1, buf1, si1)
            pltpu.make_async_copy(
                imp_hbm.at[b, pl.ds(c0 * chunk, chunk)], buf0, si0
            ).wait()
            hist_chunk(buf0)

            @pl.when(g + 1 < npair)
            def _():
                in_copy(c0 + 2, buf0, si0)

            pltpu.make_async_copy(
                imp_hbm.at[b, pl.ds((c0 + 1) * chunk, chunk)], buf1, si1
            ).wait()
            hist_chunk(buf1)
            return 0

        lax.fori_loop(0, npair, pair_body, 0)

        def scan_body(i, carry):
            cum, bstar, cabove = carry
            bi = 255 - i
            s = jnp.sum(hist[bi])
            newcum = cum + s
            hit = (cum < k_rem) & (newcum >= k_rem)
            return (
                newcum,
                jnp.where(hit, bi, bstar),
                jnp.where(hit, cum, cabove),
            )

        _, bstar, cabove = plsc.parallel_loop(
            0, 256, unroll=4,
            carry=(jnp.int32(0), jnp.int32(0), jnp.int32(0)),
        )(scan_body)
        prefix = (prefix << 8) | bstar
        k_rem = k_rem - cabove

    t_key = prefix ^ jnp.int32(_MIN32)
    need = k_rem

    def mask_chunk(buf, ob, carry):
        def mask_body(j, carry, buf=buf, ob=ob):
            rank_c, cnt_c = carry
            key = keys_at(buf, j)
            gt = key > t_key
            tie = key == t_key
            cs = plsc.cumsum(jnp.where(tie, jnp.int32(1), jnp.int32(0)))
            keep = gt | (tie & ((cs + rank_c) <= need))
            ob[pl.ds(j * 16, 16)] = jnp.where(
                keep, jnp.float32(1.0), jnp.float32(0.0)
            )
            return (
                rank_c + plsc.all_reduce_population_count(tie),
                cnt_c + plsc.all_reduce_population_count(keep),
            )

        return plsc.parallel_loop(0, nvec, unroll=8, carry=carry)(mask_body)

    def out_copy(c, ob, sem):
        return pltpu.async_copy(ob, mask_hbm.at[b, pl.ds(c * chunk, chunk)], sem)

    def out_wait(c, ob, sem):
        pltpu.make_async_copy(ob, mask_hbm.at[b, pl.ds(c * chunk, chunk)], sem).wait()

    in_copy(0, buf0, si0)

    def mask_pair_body(g, carry):
        c0 = g * 2
        in_copy(c0 + 1, buf1, si1)
        pltpu.make_async_copy(
            imp_hbm.at[b, pl.ds(c0 * chunk, chunk)], buf0, si0
        ).wait()

        @pl.when(g >= 1)
        def _():
            out_wait(c0 - 2, ob0, so0)

        carry = mask_chunk(buf0, ob0, carry)
        out_copy(c0, ob0, so0)

        @pl.when(g + 1 < npair)
        def _():
            in_copy(c0 + 2, buf0, si0)

        pltpu.make_async_copy(
            imp_hbm.at[b, pl.ds((c0 + 1) * chunk, chunk)], buf1, si1
        ).wait()

        @pl.when(g >= 1)
        def _():
            out_wait(c0 - 1, ob1, so1)

        carry = mask_chunk(buf1, ob1, carry)
        out_copy(c0 + 1, ob1, so1)
        return carry

    rank_c, cnt_c = lax.fori_loop(
        0, npair, mask_pair_body,
        (jnp.zeros((16,), jnp.int32), jnp.zeros((16,), jnp.int32)),
    )
    out_wait(nch - 2, ob0, so0)
    out_wait(nch - 1, ob1, so1)

    cbuf[pl.ds(0, 16)] = cnt_c.astype(jnp.float32)
    pltpu.sync_copy(cbuf, cnt_hbm.at[b])


@jax.jit
def kernel(imp):
    B, H, W = imp.shape
    n = H * W
    k = max(1, int(_RATE * n))
    chunk = 16384
    mesh = plsc.VectorSubcoreMesh(core_axis_name="c", subcore_axis_name="s")
    sc_call = pl.kernel(
        functools.partial(_sc_body, k, n, chunk),
        out_type=[
            jax.ShapeDtypeStruct((B, n), jnp.float32),
            jax.ShapeDtypeStruct((B, 16), jnp.float32),
        ],
        mesh=mesh,
        compiler_params=pltpu.CompilerParams(needs_layout_passes=False),
        scratch_types=[
            pltpu.VMEM((chunk,), jnp.float32),
            pltpu.VMEM((chunk,), jnp.float32),
            pltpu.VMEM((chunk,), jnp.float32),
            pltpu.VMEM((chunk,), jnp.float32),
            pltpu.VMEM((256, 16), jnp.int32),
            pltpu.VMEM((16,), jnp.float32),
            pltpu.SemaphoreType.DMA,
            pltpu.SemaphoreType.DMA,
            pltpu.SemaphoreType.DMA,
            pltpu.SemaphoreType.DMA,
        ],
    )
    mask2d, cnt = sc_call(imp.reshape(B, n))
    mean = jnp.sum(cnt[:, 0]) / jnp.float32(B * n)
    return mask2d.reshape(B, 1, H, W), mean


# speedup vs baseline: 5.9571x; 1.2834x over previous
import functools

import jax
import jax.numpy as jnp
from jax import lax
from jax.experimental import pallas as pl
from jax.experimental.pallas import tpu as pltpu
from jax.experimental.pallas import tpu_sc as plsc

_RATE = 0.75
_MIN32 = -(2**31)


def _sc_body(
    k, h, w, rows,
    imp_hbm, mask_hbm, cnt_hbm,
    buf0, buf1, ob0, ob1, hist, cbuf,
    si0, si1, so0, so1,
):
    b = lax.axis_index("s") * 2 + lax.axis_index("c")
    lane = lax.iota(jnp.int32, 16)
    ones = jnp.ones((16,), jnp.int32)
    nch = h // rows
    npair = nch // 2
    nvec = rows * w // 16
    vrow = w // 16

    def in_copy(c, buf, sem):
        return pltpu.async_copy(
            imp_hbm.at[b, pl.ds(c * rows, rows), :], buf, sem
        )

    def in_wait(c, buf, sem):
        pltpu.make_async_copy(
            imp_hbm.at[b, pl.ds(c * rows, rows), :], buf, sem
        ).wait()

    def keys_at(buf, j):
        x = buf[j // vrow, pl.ds((j % vrow) * 16, 16)]
        i32 = lax.bitcast_convert_type(x, jnp.int32)
        return i32 ^ ((i32 >> 31) & jnp.int32(0x7FFFFFFF))

    prefix = jnp.int32(0)
    k_rem = jnp.int32(k)
    for p in range(4):
        shift_b = 24 - 8 * p

        def zero_body(i, _):
            hist[i] = jnp.zeros((16,), jnp.int32)
            return 0

        lax.fori_loop(0, 256, zero_body, 0, unroll=8)

        def hist_chunk(buf):
            @plsc.parallel_loop(0, nvec, unroll=8)
            def _(j, buf=buf, p=p, shift_b=shift_b):
                ukey = keys_at(buf, j) ^ jnp.int32(_MIN32)
                if shift_b:
                    bucket = lax.shift_right_logical(ukey, shift_b) & 0xFF
                else:
                    bucket = ukey & 0xFF
                if p == 0:
                    plsc.addupdate_scatter(hist, [bucket, lane], ones)
                else:
                    hi = lax.shift_right_logical(ukey, shift_b + 8)
                    plsc.addupdate_scatter(
                        hist, [bucket, lane], ones, mask=hi == prefix
                    )

        in_copy(0, buf0, si0)

        def pair_body(g, _):
            c0 = g * 2
            in_copy(c0 + 1, buf1, si1)
            in_wait(c0, buf0, si0)
            hist_chunk(buf0)

            @pl.when(g + 1 < npair)
            def _():
                in_copy(c0 + 2, buf0, si0)

            in_wait(c0 + 1, buf1, si1)
            hist_chunk(buf1)
            return 0

        lax.fori_loop(0, npair, pair_body, 0)

        def scan_body(i, carry):
            cum, bstar, cabove = carry
            bi = 255 - i
            s = jnp.sum(hist[bi])
            newcum = cum + s
            hit = (cum < k_rem) & (newcum >= k_rem)
            return (
                newcum,
                jnp.where(hit, bi, bstar),
                jnp.where(hit, cum, cabove),
            )

        _, bstar, cabove = plsc.parallel_loop(
            0, 256, unroll=4,
            carry=(jnp.int32(0), jnp.int32(0), jnp.int32(0)),
        )(scan_body)
        prefix = (prefix << 8) | bstar
        k_rem = k_rem - cabove

    t_key = prefix ^ jnp.int32(_MIN32)
    need = k_rem

    def mask_chunk(buf, ob, carry):
        def mask_body(j, carry, buf=buf, ob=ob):
            rank_c, cnt_c = carry
            key = keys_at(buf, j)
            gt = key > t_key
            tie = key == t_key
            cs = plsc.cumsum(jnp.where(tie, jnp.int32(1), jnp.int32(0)))
            keep = gt | (tie & ((cs + rank_c) <= need))
            ob[j // vrow, pl.ds((j % vrow) * 16, 16)] = jnp.where(
                keep, jnp.float32(1.0), jnp.float32(0.0)
            )
            return (
                rank_c + plsc.all_reduce_population_count(tie),
                cnt_c + plsc.all_reduce_population_count(keep),
            )

        return plsc.parallel_loop(0, nvec, unroll=8, carry=carry)(mask_body)

    def out_copy(c, ob, sem):
        return pltpu.async_copy(
            ob, mask_hbm.at[b, pl.ds(c * rows, rows), :], sem
        )

    def out_wait(c, ob, sem):
        pltpu.make_async_copy(
            ob, mask_hbm.at[b, pl.ds(c * rows, rows), :], sem
        ).wait()

    in_copy(0, buf0, si0)

    def mask_pair_body(g, carry):
        c0 = g * 2
        in_copy(c0 + 1, buf1, si1)
        in_wait(c0, buf0, si0)

        @pl.when(g >= 1)
        def _():
            out_wait(c0 - 2, ob0, so0)

        carry = mask_chunk(buf0, ob0, carry)
        out_copy(c0, ob0, so0)

        @pl.when(g + 1 < npair)
        def _():
            in_copy(c0 + 2, buf0, si0)

        in_wait(c0 + 1, buf1, si1)

        @pl.when(g >= 1)
        def _():
            out_wait(c0 - 1, ob1, so1)

        carry = mask_chunk(buf1, ob1, carry)
        out_copy(c0 + 1, ob1, so1)
        return carry

    rank_c, cnt_c = lax.fori_loop(
        0, npair, mask_pair_body,
        (jnp.zeros((16,), jnp.int32), jnp.zeros((16,), jnp.int32)),
    )
    out_wait(nch - 2, ob0, so0)
    out_wait(nch - 1, ob1, so1)

    cbuf[pl.ds(0, 16)] = cnt_c.astype(jnp.float32)
    pltpu.sync_copy(cbuf, cnt_hbm.at[b])


@jax.jit
def kernel(imp):
    B, H, W = imp.shape
    n = H * W
    k = max(1, int(_RATE * n))
    rows = 16384 // W
    mesh = plsc.VectorSubcoreMesh(core_axis_name="c", subcore_axis_name="s")
    sc_call = pl.kernel(
        functools.partial(_sc_body, k, H, W, rows),
        out_type=[
            jax.ShapeDtypeStruct((B, H, W), jnp.float32),
            jax.ShapeDtypeStruct((B, 16), jnp.float32),
        ],
        mesh=mesh,
        compiler_params=pltpu.CompilerParams(
            needs_layout_passes=False, use_tc_tiling_on_sc=True
        ),
        scratch_types=[
            pltpu.VMEM((16384 // 512, 512), jnp.float32),
            pltpu.VMEM((16384 // 512, 512), jnp.float32),
            pltpu.VMEM((16384 // 512, 512), jnp.float32),
            pltpu.VMEM((16384 // 512, 512), jnp.float32),
            pltpu.VMEM((256, 16), jnp.int32),
            pltpu.VMEM((16,), jnp.float32),
            pltpu.SemaphoreType.DMA,
            pltpu.SemaphoreType.DMA,
            pltpu.SemaphoreType.DMA,
            pltpu.SemaphoreType.DMA,
        ],
    )
    mask3d, cnt = sc_call(imp)
    mean = jnp.sum(cnt[:, 0]) / jnp.float32(B * n)
    return mask3d[:, None, :, :], mean


# speedup vs baseline: 6.5879x; 1.1059x over previous
import functools

import jax
import jax.numpy as jnp
from jax import lax
from jax.experimental import pallas as pl
from jax.experimental.pallas import tpu as pltpu
from jax.experimental.pallas import tpu_sc as plsc

_RATE = 0.75
_MIN32 = -(2**31)


def _sc_body(
    k, h, w, rows,
    imp_hbm, mask_hbm, cnt_hbm,
    buf0, buf1, ob0, ob1, hist, cbuf,
    si0, si1, so0, so1,
):
    b = lax.axis_index("s") * 2 + lax.axis_index("c")
    lane = lax.iota(jnp.int32, 16)
    ones = jnp.ones((16,), jnp.int32)
    nch = h // rows
    npair = nch // 2
    nvec = rows * w // 16
    vrow = w // 16

    def in_copy(c, buf, sem):
        return pltpu.async_copy(
            imp_hbm.at[b, pl.ds(c * rows, rows), :], buf, sem
        )

    def in_wait(c, buf, sem):
        pltpu.make_async_copy(
            imp_hbm.at[b, pl.ds(c * rows, rows), :], buf, sem
        ).wait()

    def keys_at(buf, j):
        x = buf[j // vrow, pl.ds((j % vrow) * 16, 16)]
        i32 = lax.bitcast_convert_type(x, jnp.int32)
        return i32 ^ ((i32 >> 31) & jnp.int32(0x7FFFFFFF))

    prefix = jnp.int32(0)
    k_rem = jnp.int32(k)
    for p, (shift_b, nbits, prior_shift) in enumerate(
        [(21, 11, None), (10, 11, 21), (0, 10, 10)]
    ):
        nbins = 1 << nbits

        def zero_body(i, _):
            hist[i >> 3, pl.ds((i & 7) * 16, 16)] = jnp.zeros((16,), jnp.int32)
            return 0

        lax.fori_loop(0, nbins, zero_body, 0, unroll=8)

        def hist_chunk(buf):
            @plsc.parallel_loop(0, nvec, unroll=8)
            def _(j, buf=buf, shift_b=shift_b, nbins=nbins, prior_shift=prior_shift):
                ukey = keys_at(buf, j) ^ jnp.int32(_MIN32)
                if shift_b:
                    bucket = lax.shift_right_logical(ukey, shift_b) & (nbins - 1)
                else:
                    bucket = ukey & (nbins - 1)
                hrow = lax.shift_right_logical(bucket, 3)
                hcol = ((bucket & 7) << 4) | lane
                if prior_shift is None:
                    plsc.addupdate_scatter(hist, [hrow, hcol], ones)
                else:
                    hi = lax.shift_right_logical(ukey, prior_shift)
                    plsc.addupdate_scatter(
                        hist, [hrow, hcol], ones, mask=hi == prefix
                    )

        in_copy(0, buf0, si0)

        def pair_body(g, _):
            c0 = g * 2
            in_copy(c0 + 1, buf1, si1)
            in_wait(c0, buf0, si0)
            hist_chunk(buf0)

            @pl.when(g + 1 < npair)
            def _():
                in_copy(c0 + 2, buf0, si0)

            in_wait(c0 + 1, buf1, si1)
            hist_chunk(buf1)
            return 0

        lax.fori_loop(0, npair, pair_body, 0)

        def scan_body(i, carry, nbins=nbins):
            cum, bstar, cabove = carry
            bi = nbins - 1 - i
            s = jnp.sum(hist[bi >> 3, pl.ds((bi & 7) * 16, 16)])
            newcum = cum + s
            hit = (cum < k_rem) & (newcum >= k_rem)
            return (
                newcum,
                jnp.where(hit, bi, bstar),
                jnp.where(hit, cum, cabove),
            )

        _, bstar, cabove = plsc.parallel_loop(
            0, nbins, unroll=4,
            carry=(jnp.int32(0), jnp.int32(0), jnp.int32(0)),
        )(scan_body)
        prefix = (prefix << nbits) | bstar
        k_rem = k_rem - cabove

    t_key = prefix ^ jnp.int32(_MIN32)
    need = k_rem

    def mask_chunk(buf, ob, carry):
        def mask_body(j, carry, buf=buf, ob=ob):
            rank_c, cnt_c = carry
            key = keys_at(buf, j)
            gt = key > t_key
            tie = key == t_key
            cs = plsc.cumsum(jnp.where(tie, jnp.int32(1), jnp.int32(0)))
            keep = gt | (tie & ((cs + rank_c) <= need))
            ob[j // vrow, pl.ds((j % vrow) * 16, 16)] = jnp.where(
                keep, jnp.float32(1.0), jnp.float32(0.0)
            )
            return (
                rank_c + plsc.all_reduce_population_count(tie),
                cnt_c + plsc.all_reduce_population_count(keep),
            )

        return plsc.parallel_loop(0, nvec, unroll=8, carry=carry)(mask_body)

    def out_copy(c, ob, sem):
        return pltpu.async_copy(
            ob, mask_hbm.at[b, pl.ds(c * rows, rows), :], sem
        )

    def out_wait(c, ob, sem):
        pltpu.make_async_copy(
            ob, mask_hbm.at[b, pl.ds(c * rows, rows), :], sem
        ).wait()

    in_copy(0, buf0, si0)

    def mask_pair_body(g, carry):
        c0 = g * 2
        in_copy(c0 + 1, buf1, si1)
        in_wait(c0, buf0, si0)

        @pl.when(g >= 1)
        def _():
            out_wait(c0 - 2, ob0, so0)

        carry = mask_chunk(buf0, ob0, carry)
        out_copy(c0, ob0, so0)

        @pl.when(g + 1 < npair)
        def _():
            in_copy(c0 + 2, buf0, si0)

        in_wait(c0 + 1, buf1, si1)

        @pl.when(g >= 1)
        def _():
            out_wait(c0 - 1, ob1, so1)

        carry = mask_chunk(buf1, ob1, carry)
        out_copy(c0 + 1, ob1, so1)
        return carry

    rank_c, cnt_c = lax.fori_loop(
        0, npair, mask_pair_body,
        (jnp.zeros((16,), jnp.int32), jnp.zeros((16,), jnp.int32)),
    )
    out_wait(nch - 2, ob0, so0)
    out_wait(nch - 1, ob1, so1)

    cbuf[pl.ds(0, 16)] = cnt_c.astype(jnp.float32)
    pltpu.sync_copy(cbuf, cnt_hbm.at[b])


@jax.jit
def kernel(imp):
    B, H, W = imp.shape
    n = H * W
    k = max(1, int(_RATE * n))
    rows = 16384 // W
    mesh = plsc.VectorSubcoreMesh(core_axis_name="c", subcore_axis_name="s")
    sc_call = pl.kernel(
        functools.partial(_sc_body, k, H, W, rows),
        out_type=[
            jax.ShapeDtypeStruct((B, H, W), jnp.float32),
            jax.ShapeDtypeStruct((B, 16), jnp.float32),
        ],
        mesh=mesh,
        compiler_params=pltpu.CompilerParams(
            needs_layout_passes=False, use_tc_tiling_on_sc=True
        ),
        scratch_types=[
            pltpu.VMEM((16384 // 512, 512), jnp.float32),
            pltpu.VMEM((16384 // 512, 512), jnp.float32),
            pltpu.VMEM((16384 // 512, 512), jnp.float32),
            pltpu.VMEM((16384 // 512, 512), jnp.float32),
            pltpu.VMEM((256, 128), jnp.int32),
            pltpu.VMEM((16,), jnp.float32),
            pltpu.SemaphoreType.DMA,
            pltpu.SemaphoreType.DMA,
            pltpu.SemaphoreType.DMA,
            pltpu.SemaphoreType.DMA,
        ],
    )
    mask3d, cnt = sc_call(imp)
    mean = jnp.sum(cnt[:, 0]) / jnp.float32(B * n)
    return mask3d[:, None, :, :], mean


# speedup vs baseline: 6.7830x; 1.0296x over previous
import functools

import jax
import jax.numpy as jnp
from jax import lax
from jax.experimental import pallas as pl
from jax.experimental.pallas import tpu as pltpu
from jax.experimental.pallas import tpu_sc as plsc

_RATE = 0.75
_MIN32 = -(2**31)


def _sc_body(
    k, h, w, rows,
    imp_hbm, mask_hbm, cnt_hbm,
    buf0, buf1, ob0, ob1, hist, cbuf,
    si0, si1, so0, so1,
):
    b = lax.axis_index("s") * 2 + lax.axis_index("c")
    lane = lax.iota(jnp.int32, 16)
    ones = jnp.ones((16,), jnp.int32)
    nch = h // rows
    npair = nch // 2
    nvec = rows * w // 16
    vrow = w // 16

    def in_copy(c, buf, sem):
        return pltpu.async_copy(
            imp_hbm.at[b, pl.ds(c * rows, rows), :], buf, sem
        )

    def in_wait(c, buf, sem):
        pltpu.make_async_copy(
            imp_hbm.at[b, pl.ds(c * rows, rows), :], buf, sem
        ).wait()

    def keys_at(buf, j):
        x = buf[j // vrow, pl.ds((j % vrow) * 16, 16)]
        i32 = lax.bitcast_convert_type(x, jnp.int32)
        return i32 ^ ((i32 >> 31) & jnp.int32(0x7FFFFFFF))

    def ukeys_at(buf, j):
        x = buf[j // vrow, pl.ds((j % vrow) * 16, 16)]
        i32 = lax.bitcast_convert_type(x, jnp.int32)
        return i32 ^ ((i32 >> 31) | jnp.int32(_MIN32))

    prefix = jnp.int32(0)
    k_rem = jnp.int32(k)
    for p, (shift_b, nbits, prior_shift) in enumerate(
        [(21, 11, None), (10, 11, 21), (0, 10, 10)]
    ):
        nbins = 1 << nbits

        def zero_body(i, _):
            hist[i >> 3, pl.ds((i & 7) * 16, 16)] = jnp.zeros((16,), jnp.int32)
            return 0

        lax.fori_loop(0, nbins, zero_body, 0, unroll=8)

        def hist_chunk(buf):
            @plsc.parallel_loop(0, nvec, unroll=8)
            def _(j, buf=buf, shift_b=shift_b, nbins=nbins, prior_shift=prior_shift):
                ukey = ukeys_at(buf, j)
                if shift_b:
                    bucket = lax.shift_right_logical(ukey, shift_b) & (nbins - 1)
                else:
                    bucket = ukey & (nbins - 1)
                hrow = lax.shift_right_logical(bucket, 3)
                hcol = ((bucket & 7) << 4) | lane
                if prior_shift is None:
                    plsc.addupdate_scatter(hist, [hrow, hcol], ones)
                else:
                    hi = lax.shift_right_logical(ukey, prior_shift)
                    plsc.addupdate_scatter(
                        hist, [hrow, hcol], ones, mask=hi == prefix
                    )

        in_copy(0, buf0, si0)

        def pair_body(g, _):
            c0 = g * 2
            in_copy(c0 + 1, buf1, si1)
            in_wait(c0, buf0, si0)
            hist_chunk(buf0)

            @pl.when(g + 1 < npair)
            def _():
                in_copy(c0 + 2, buf0, si0)

            in_wait(c0 + 1, buf1, si1)
            hist_chunk(buf1)
            return 0

        lax.fori_loop(0, npair, pair_body, 0)

        def scan_body(i, carry, nbins=nbins):
            cum, bstar, cabove = carry
            bi = nbins - 1 - i
            s = jnp.sum(hist[bi >> 3, pl.ds((bi & 7) * 16, 16)])
            newcum = cum + s
            hit = (cum < k_rem) & (newcum >= k_rem)
            return (
                newcum,
                jnp.where(hit, bi, bstar),
                jnp.where(hit, cum, cabove),
            )

        _, bstar, cabove = plsc.parallel_loop(
            0, nbins, unroll=4,
            carry=(jnp.int32(0), jnp.int32(0), jnp.int32(0)),
        )(scan_body)
        prefix = (prefix << nbits) | bstar
        k_rem = k_rem - cabove

    t_key = prefix ^ jnp.int32(_MIN32)
    need = k_rem

    def mask_chunk(buf, ob, carry):
        def mask_body(j, carry, buf=buf, ob=ob):
            rank_c, cnt_c = carry
            key = keys_at(buf, j)
            gt = key > t_key
            tie = key == t_key
            cs = plsc.cumsum(jnp.where(tie, jnp.int32(1), jnp.int32(0)))
            keep = gt | (tie & ((cs + rank_c) <= need))
            ob[j // vrow, pl.ds((j % vrow) * 16, 16)] = jnp.where(
                keep, jnp.float32(1.0), jnp.float32(0.0)
            )
            return (
                rank_c + plsc.all_reduce_population_count(tie),
                cnt_c + plsc.all_reduce_population_count(keep),
            )

        return plsc.parallel_loop(0, nvec, unroll=8, carry=carry)(mask_body)

    def out_copy(c, ob, sem):
        return pltpu.async_copy(
            ob, mask_hbm.at[b, pl.ds(c * rows, rows), :], sem
        )

    def out_wait(c, ob, sem):
        pltpu.make_async_copy(
            ob, mask_hbm.at[b, pl.ds(c * rows, rows), :], sem
        ).wait()

    in_copy(0, buf0, si0)

    def mask_pair_body(g, carry):
        c0 = g * 2
        in_copy(c0 + 1, buf1, si1)
        in_wait(c0, buf0, si0)

        @pl.when(g >= 1)
        def _():
            out_wait(c0 - 2, ob0, so0)

        carry = mask_chunk(buf0, ob0, carry)
        out_copy(c0, ob0, so0)

        @pl.when(g + 1 < npair)
        def _():
            in_copy(c0 + 2, buf0, si0)

        in_wait(c0 + 1, buf1, si1)

        @pl.when(g >= 1)
        def _():
            out_wait(c0 - 1, ob1, so1)

        carry = mask_chunk(buf1, ob1, carry)
        out_copy(c0 + 1, ob1, so1)
        return carry

    rank_c, cnt_c = lax.fori_loop(
        0, npair, mask_pair_body,
        (jnp.zeros((16,), jnp.int32), jnp.zeros((16,), jnp.int32)),
    )
    out_wait(nch - 2, ob0, so0)
    out_wait(nch - 1, ob1, so1)

    cbuf[pl.ds(0, 16)] = cnt_c.astype(jnp.float32)
    pltpu.sync_copy(cbuf, cnt_hbm.at[b])


@jax.jit
def kernel(imp):
    B, H, W = imp.shape
    n = H * W
    k = max(1, int(_RATE * n))
    rows = 16384 // W
    mesh = plsc.VectorSubcoreMesh(core_axis_name="c", subcore_axis_name="s")
    sc_call = pl.kernel(
        functools.partial(_sc_body, k, H, W, rows),
        out_type=[
            jax.ShapeDtypeStruct((B, H, W), jnp.float32),
            jax.ShapeDtypeStruct((B, 16), jnp.float32),
        ],
        mesh=mesh,
        compiler_params=pltpu.CompilerParams(
            needs_layout_passes=False, use_tc_tiling_on_sc=True
        ),
        scratch_types=[
            pltpu.VMEM((16384 // 512, 512), jnp.float32),
            pltpu.VMEM((16384 // 512, 512), jnp.float32),
            pltpu.VMEM((16384 // 512, 512), jnp.float32),
            pltpu.VMEM((16384 // 512, 512), jnp.float32),
            pltpu.VMEM((256, 128), jnp.int32),
            pltpu.VMEM((16,), jnp.float32),
            pltpu.SemaphoreType.DMA,
            pltpu.SemaphoreType.DMA,
            pltpu.SemaphoreType.DMA,
            pltpu.SemaphoreType.DMA,
        ],
    )
    mask3d, cnt = sc_call(imp)
    mean = jnp.sum(cnt[:, 0]) / jnp.float32(B * n)
    return mask3d[:, None, :, :], mean


# speedup vs baseline: 7.0204x; 1.0350x over previous
import functools

import jax
import jax.numpy as jnp
from jax import lax
from jax.experimental import pallas as pl
from jax.experimental.pallas import tpu as pltpu
from jax.experimental.pallas import tpu_sc as plsc

_RATE = 0.75
_MIN32 = -(2**31)


def _sc_body(
    k, h, w, rows,
    imp_hbm, mask_hbm, cnt_hbm,
    buf0, buf1, ob0, ob1, hist, cbuf,
    si0, si1, so0, so1,
):
    b = lax.axis_index("s") * 2 + lax.axis_index("c")
    lane = lax.iota(jnp.int32, 16)
    ones = jnp.ones((16,), jnp.int32)
    nch = h // rows
    npair = nch // 2
    nvec = rows * w // 16
    vrow = w // 16

    def in_copy(c, buf, sem):
        return pltpu.async_copy(
            imp_hbm.at[b, pl.ds(c * rows, rows), :], buf, sem
        )

    def in_wait(c, buf, sem):
        pltpu.make_async_copy(
            imp_hbm.at[b, pl.ds(c * rows, rows), :], buf, sem
        ).wait()

    def keys_at(buf, j):
        x = buf[j // vrow, pl.ds((j % vrow) * 16, 16)]
        i32 = lax.bitcast_convert_type(x, jnp.int32)
        return i32 ^ ((i32 >> 31) & jnp.int32(0x7FFFFFFF))

    def ukeys_at(buf, j):
        x = buf[j // vrow, pl.ds((j % vrow) * 16, 16)]
        i32 = lax.bitcast_convert_type(x, jnp.int32)
        return i32 ^ ((i32 >> 31) | jnp.int32(_MIN32))

    prefix = jnp.int32(0)
    k_rem = jnp.int32(k)
    for p, (shift_b, nbits, prior_shift) in enumerate(
        [(21, 11, None), (10, 11, 21), (0, 10, 10)]
    ):
        nbins = 1 << nbits

        def zero_body(i, _):
            hist[i >> 3, pl.ds((i & 7) * 16, 16)] = jnp.zeros((16,), jnp.int32)
            return 0

        lax.fori_loop(0, nbins, zero_body, 0, unroll=8)

        def hist_chunk(buf):
            @plsc.parallel_loop(0, nvec, unroll=8)
            def _(j, buf=buf, shift_b=shift_b, nbins=nbins, prior_shift=prior_shift):
                ukey = ukeys_at(buf, j)
                if shift_b:
                    bucket = lax.shift_right_logical(ukey, shift_b) & (nbins - 1)
                else:
                    bucket = ukey & (nbins - 1)
                hrow = lax.shift_right_logical(bucket, 3)
                hcol = ((bucket & 7) << 4) | lane
                if prior_shift is None:
                    plsc.addupdate_scatter(hist, [hrow, hcol], ones)
                else:
                    hi = lax.shift_right_logical(ukey, prior_shift)
                    plsc.addupdate_scatter(
                        hist, [hrow, hcol], ones, mask=hi == prefix
                    )

        if p == 0:
            in_copy(0, buf0, si0)

        def pair_body(g, _):
            c0 = g * 2
            in_copy(c0 + 1, buf1, si1)
            in_wait(c0, buf0, si0)
            hist_chunk(buf0)

            @pl.when(g + 1 < npair)
            def _():
                in_copy(c0 + 2, buf0, si0)

            in_wait(c0 + 1, buf1, si1)
            hist_chunk(buf1)
            return 0

        lax.fori_loop(0, npair, pair_body, 0)
        in_copy(0, buf0, si0)

        def scan_body(i, carry, nbins=nbins):
            cum, bstar, cabove = carry
            bi = nbins - 1 - i
            s = jnp.sum(hist[bi >> 3, pl.ds((bi & 7) * 16, 16)])
            newcum = cum + s
            hit = (cum < k_rem) & (newcum >= k_rem)
            return (
                newcum,
                jnp.where(hit, bi, bstar),
                jnp.where(hit, cum, cabove),
            )

        _, bstar, cabove = plsc.parallel_loop(
            0, nbins, unroll=4,
            carry=(jnp.int32(0), jnp.int32(0), jnp.int32(0)),
        )(scan_body)
        prefix = (prefix << nbits) | bstar
        k_rem = k_rem - cabove

    t_key = prefix ^ jnp.int32(_MIN32)
    need = k_rem

    def mask_chunk(buf, ob, carry):
        def mask_body(j, carry, buf=buf, ob=ob):
            rank_c, cnt_c = carry
            key = keys_at(buf, j)
            gt = key > t_key
            tie = key == t_key
            cs = plsc.cumsum(jnp.where(tie, jnp.int32(1), jnp.int32(0)))
            keep = gt | (tie & ((cs + rank_c) <= need))
            ob[j // vrow, pl.ds((j % vrow) * 16, 16)] = jnp.where(
                keep, jnp.float32(1.0), jnp.float32(0.0)
            )
            return (
                rank_c + plsc.all_reduce_population_count(tie),
                cnt_c + plsc.all_reduce_population_count(keep),
            )

        return plsc.parallel_loop(0, nvec, unroll=8, carry=carry)(mask_body)

    def out_copy(c, ob, sem):
        return pltpu.async_copy(
            ob, mask_hbm.at[b, pl.ds(c * rows, rows), :], sem
        )

    def out_wait(c, ob, sem):
        pltpu.make_async_copy(
            ob, mask_hbm.at[b, pl.ds(c * rows, rows), :], sem
        ).wait()

    def mask_pair_body(g, carry):
        c0 = g * 2
        in_copy(c0 + 1, buf1, si1)
        in_wait(c0, buf0, si0)

        @pl.when(g >= 1)
        def _():
            out_wait(c0 - 2, ob0, so0)

        carry = mask_chunk(buf0, ob0, carry)
        out_copy(c0, ob0, so0)

        @pl.when(g + 1 < npair)
        def _():
            in_copy(c0 + 2, buf0, si0)

        in_wait(c0 + 1, buf1, si1)

        @pl.when(g >= 1)
        def _():
            out_wait(c0 - 1, ob1, so1)

        carry = mask_chunk(buf1, ob1, carry)
        out_copy(c0 + 1, ob1, so1)
        return carry

    rank_c, cnt_c = lax.fori_loop(
        0, npair, mask_pair_body,
        (jnp.zeros((16,), jnp.int32), jnp.zeros((16,), jnp.int32)),
    )
    out_wait(nch - 2, ob0, so0)
    out_wait(nch - 1, ob1, so1)

    cbuf[pl.ds(0, 16)] = cnt_c.astype(jnp.float32)
    pltpu.sync_copy(cbuf, cnt_hbm.at[b])


@jax.jit
def kernel(imp):
    B, H, W = imp.shape
    n = H * W
    k = max(1, int(_RATE * n))
    rows = 16384 // W
    mesh = plsc.VectorSubcoreMesh(core_axis_name="c", subcore_axis_name="s")
    sc_call = pl.kernel(
        functools.partial(_sc_body, k, H, W, rows),
        out_type=[
            jax.ShapeDtypeStruct((B, H, W), jnp.float32),
            jax.ShapeDtypeStruct((B, 16), jnp.float32),
        ],
        mesh=mesh,
        compiler_params=pltpu.CompilerParams(
            needs_layout_passes=False, use_tc_tiling_on_sc=True
        ),
        scratch_types=[
            pltpu.VMEM((16384 // 512, 512), jnp.float32),
            pltpu.VMEM((16384 // 512, 512), jnp.float32),
            pltpu.VMEM((16384 // 512, 512), jnp.float32),
            pltpu.VMEM((16384 // 512, 512), jnp.float32),
            pltpu.VMEM((256, 128), jnp.int32),
            pltpu.VMEM((16,), jnp.float32),
            pltpu.SemaphoreType.DMA,
            pltpu.SemaphoreType.DMA,
            pltpu.SemaphoreType.DMA,
            pltpu.SemaphoreType.DMA,
        ],
    )
    mask3d, cnt = sc_call(imp)
    mean = jnp.sum(cnt[:, 0]) / jnp.float32(B * n)
    return mask3d[:, None, :, :], mean


# speedup vs baseline: 7.0226x; 1.0003x over previous
import functools

import jax
import jax.numpy as jnp
from jax import lax
from jax.experimental import pallas as pl
from jax.experimental.pallas import tpu as pltpu
from jax.experimental.pallas import tpu_sc as plsc

_RATE = 0.75
_MIN32 = -(2**31)


def _sc_body(
    k, h, w, rows,
    imp_hbm, mask_hbm, cnt_hbm,
    buf0, buf1, ob0, ob1, hist, cbuf,
    si0, si1, so0, so1,
):
    b = lax.axis_index("s") * 2 + lax.axis_index("c")
    lane = lax.iota(jnp.int32, 16)
    ones = jnp.ones((16,), jnp.int32)
    nch = h // rows
    npair = nch // 2
    nvec = rows * w // 16
    vrow = w // 16

    def in_copy(c, buf, sem):
        return pltpu.async_copy(
            imp_hbm.at[b, pl.ds(c * rows, rows), :], buf, sem
        )

    def in_wait(c, buf, sem):
        pltpu.make_async_copy(
            imp_hbm.at[b, pl.ds(c * rows, rows), :], buf, sem
        ).wait()

    def keys_at(buf, j):
        x = buf[j // vrow, pl.ds((j % vrow) * 16, 16)]
        i32 = lax.bitcast_convert_type(x, jnp.int32)
        return i32 ^ ((i32 >> 31) & jnp.int32(0x7FFFFFFF))

    def ukeys_at(buf, j):
        x = buf[j // vrow, pl.ds((j % vrow) * 16, 16)]
        i32 = lax.bitcast_convert_type(x, jnp.int32)
        return i32 ^ ((i32 >> 31) | jnp.int32(_MIN32))

    prefix = jnp.int32(0)
    k_rem = jnp.int32(k)
    for p, (shift_b, nbits, prior_shift) in enumerate(
        [(21, 11, None), (10, 11, 21), (0, 10, 10)]
    ):
        nbins = 1 << nbits

        def zero_body(i, _):
            hist[i >> 3, pl.ds((i & 7) * 16, 16)] = jnp.zeros((16,), jnp.int32)
            return 0

        lax.fori_loop(0, nbins, zero_body, 0, unroll=8)

        def hist_chunk(buf):
            @plsc.parallel_loop(0, nvec, unroll=8)
            def _(j, buf=buf, shift_b=shift_b, nbins=nbins, prior_shift=prior_shift):
                ukey = ukeys_at(buf, j)
                if shift_b:
                    bucket = lax.shift_right_logical(ukey, shift_b) & (nbins - 1)
                else:
                    bucket = ukey & (nbins - 1)
                hrow = lax.shift_right_logical(bucket, 3)
                hcol = ((bucket & 7) << 4) | lane
                if prior_shift is None:
                    plsc.addupdate_scatter(hist, [hrow, hcol], ones)
                else:
                    hi = lax.shift_right_logical(ukey, prior_shift)
                    plsc.addupdate_scatter(
                        hist, [hrow, hcol], ones, mask=hi == prefix
                    )

        if p == 0:
            in_copy(0, buf0, si0)

        def pair_body(g, _):
            c0 = g * 2
            in_copy(c0 + 1, buf1, si1)
            in_wait(c0, buf0, si0)
            hist_chunk(buf0)

            @pl.when(g + 1 < npair)
            def _():
                in_copy(c0 + 2, buf0, si0)

            in_wait(c0 + 1, buf1, si1)
            hist_chunk(buf1)
            return 0

        lax.fori_loop(0, npair, pair_body, 0)
        in_copy(0, buf0, si0)

        def scan_body(i, carry, nbins=nbins):
            cum, bstar, cabove = carry
            bi = nbins - 1 - i
            s = jnp.sum(hist[bi >> 3, pl.ds((bi & 7) * 16, 16)])
            newcum = cum + s
            hit = (cum < k_rem) & (newcum >= k_rem)
            return (
                newcum,
                jnp.where(hit, bi, bstar),
                jnp.where(hit, cum, cabove),
            )

        _, bstar, cabove = plsc.parallel_loop(
            0, nbins, unroll=4,
            carry=(jnp.int32(0), jnp.int32(0), jnp.int32(0)),
        )(scan_body)
        prefix = (prefix << nbits) | bstar
        k_rem = k_rem - cabove

    t_key = prefix ^ jnp.int32(_MIN32)
    need = k_rem

    def mask_chunk(buf, ob, carry):
        def mask_body(j, carry, buf=buf, ob=ob):
            rank_c, cnt_c = carry
            key = keys_at(buf, j)
            gt = key > t_key
            tie = key == t_key
            cs = plsc.cumsum(jnp.where(tie, jnp.int32(1), jnp.int32(0)))
            keep = gt | (tie & ((cs + rank_c) <= need))
            ob[j // vrow, pl.ds((j % vrow) * 16, 16)] = jnp.where(
                keep, jnp.float32(1.0), jnp.float32(0.0)
            )
            return (
                rank_c + plsc.all_reduce_population_count(tie),
                cnt_c + plsc.all_reduce_population_count(keep),
            )

        return plsc.parallel_loop(0, nvec, unroll=8, carry=carry)(mask_body)

    def out_copy(c, ob, sem):
        return pltpu.async_copy(
            ob, mask_hbm.at[b, pl.ds(c * rows, rows), :], sem
        )

    def out_wait(c, ob, sem):
        pltpu.make_async_copy(
            ob, mask_hbm.at[b, pl.ds(c * rows, rows), :], sem
        ).wait()

    def mask_pair_body(g, carry):
        c0 = g * 2
        in_copy(c0 + 1, buf1, si1)
        in_wait(c0, buf0, si0)

        @pl.when(g >= 1)
        def _():
            out_wait(c0 - 2, ob0, so0)

        carry = mask_chunk(buf0, ob0, carry)
        out_copy(c0, ob0, so0)

        @pl.when(g + 1 < npair)
        def _():
            in_copy(c0 + 2, buf0, si0)

        in_wait(c0 + 1, buf1, si1)

        @pl.when(g >= 1)
        def _():
            out_wait(c0 - 1, ob1, so1)

        carry = mask_chunk(buf1, ob1, carry)
        out_copy(c0 + 1, ob1, so1)
        return carry

    rank_c, cnt_c = lax.fori_loop(
        0, npair, mask_pair_body,
        (jnp.zeros((16,), jnp.int32), jnp.zeros((16,), jnp.int32)),
    )
    out_wait(nch - 2, ob0, so0)
    out_wait(nch - 1, ob1, so1)

    cbuf[pl.ds(0, 16)] = cnt_c.astype(jnp.float32)
    pltpu.sync_copy(cbuf, cnt_hbm.at[b])


@jax.jit
def kernel(imp):
    B, H, W = imp.shape
    n = H * W
    k = max(1, int(_RATE * n))
    rows = 16384 // W
    mesh = plsc.VectorSubcoreMesh(core_axis_name="c", subcore_axis_name="s")
    sc_call = pl.kernel(
        functools.partial(_sc_body, k, H, W, rows),
        out_type=[
            jax.ShapeDtypeStruct((B, H, W), jnp.float32),
            jax.ShapeDtypeStruct((B, 16), jnp.float32),
        ],
        mesh=mesh,
        compiler_params=pltpu.CompilerParams(
            needs_layout_passes=False, use_tc_tiling_on_sc=True
        ),
        scratch_types=[
            pltpu.VMEM((rows, W), jnp.float32),
            pltpu.VMEM((rows, W), jnp.float32),
            pltpu.VMEM((rows, W), jnp.float32),
            pltpu.VMEM((rows, W), jnp.float32),
            pltpu.VMEM((256, 128), jnp.int32),
            pltpu.VMEM((16,), jnp.float32),
            pltpu.SemaphoreType.DMA,
            pltpu.SemaphoreType.DMA,
            pltpu.SemaphoreType.DMA,
            pltpu.SemaphoreType.DMA,
        ],
    )
    mask3d, cnt = sc_call(imp)
    mean = jnp.sum(cnt[:, 0]) / jnp.float32(B * n)
    return mask3d[:, None, :, :], mean
